# baseline (speedup 1.0000x reference)
import functools

import jax
import jax.numpy as jnp
from jax import lax
from jax.experimental import pallas as pl
from jax.experimental.pallas import tpu as pltpu
from jax.experimental.pallas import tpu_sc as plsc

_BATCH = 4
_SEQ = 2048
_DIMS = 1024

_info = plsc.get_sparse_core_info()
_NC, _NS = _info.num_cores, _info.num_subcores
_NW = _NC * _NS
_ROWS_PER_W = _SEQ // _NW


def _make_gather_broadcast():
  mesh = plsc.VectorSubcoreMesh(core_axis_name="c", subcore_axis_name="s")

  n_chunks = 4
  rows_per_chunk = _ROWS_PER_W // n_chunks

  @functools.partial(
      pl.kernel,
      mesh=mesh,
      out_type=jax.ShapeDtypeStruct((_BATCH, _SEQ, _DIMS), jnp.float32),
      scratch_types=[
          pltpu.VMEM((_ROWS_PER_W,), jnp.int32),
          pltpu.VMEM((_ROWS_PER_W, _DIMS), jnp.float32),
          pltpu.SemaphoreType.DMA,
          pltpu.SemaphoreType.DMA,
      ],
  )
  def gather_broadcast(table_hbm, idx_hbm, out_hbm, idx_v, rows_v, sem_g,
                       sem_w):
    wid = lax.axis_index("s") * _NC + lax.axis_index("c")
    base = wid * _ROWS_PER_W
    pltpu.sync_copy(idx_hbm.at[pl.ds(base, _ROWS_PER_W)], idx_v)
    gathers = [
        pltpu.async_copy(
            table_hbm.at[idx_v.at[pl.ds(c * rows_per_chunk, rows_per_chunk)]],
            rows_v.at[pl.ds(c * rows_per_chunk, rows_per_chunk)],
            sem_g,
        )
        for c in range(n_chunks)
    ]
    writes = []
    for c in range(n_chunks):
      gathers[c].wait()
      lo = base + c * rows_per_chunk
      writes.append([
          pltpu.async_copy(
              rows_v.at[pl.ds(c * rows_per_chunk, rows_per_chunk)],
              out_hbm.at[b, pl.ds(lo, rows_per_chunk)],
              sem_w,
          )
          for b in range(2)
      ])
    hcopies = []
    for c in range(n_chunks):
      for w in writes[c]:
        w.wait()
      lo = base + c * rows_per_chunk
      hcopies += [
          pltpu.async_copy(
              out_hbm.at[b, pl.ds(lo, rows_per_chunk)],
              out_hbm.at[b + 2, pl.ds(lo, rows_per_chunk)],
              sem_g,
          )
          for b in range(2)
      ]
    for h in hcopies:
      h.wait()

  return gather_broadcast


_gather_broadcast = _make_gather_broadcast()


def kernel(x, E_absolute_position, relative_index):
  del x
  return _gather_broadcast(E_absolute_position, relative_index)


# speedup vs baseline: 16.4491x; 16.4491x over previous
import functools

import jax
import jax.numpy as jnp
from jax import lax
from jax.experimental import pallas as pl
from jax.experimental.pallas import tpu as pltpu
from jax.experimental.pallas import tpu_sc as plsc

_BATCH = 4
_SEQ = 2048
_DIMS = 1024

_info = plsc.get_sparse_core_info()
_NC, _NS = _info.num_cores, _info.num_subcores
_NW = _NC * _NS
_ROWS_PER_W = _SEQ // _NW


def _make_gather_broadcast():
  mesh = plsc.VectorSubcoreMesh(core_axis_name="c", subcore_axis_name="s")

  n_chunks = 4
  rows_per_chunk = _ROWS_PER_W // n_chunks

  @functools.partial(
      pl.kernel,
      mesh=mesh,
      out_type=jax.ShapeDtypeStruct((_BATCH, _SEQ, _DIMS), jnp.float32),
      scratch_types=[
          pltpu.VMEM((_ROWS_PER_W,), jnp.int32),
          pltpu.VMEM((_ROWS_PER_W, _DIMS), jnp.float32),
          pltpu.SemaphoreType.DMA,
          pltpu.SemaphoreType.DMA,
      ],
  )
  def gather_broadcast(table_hbm, idx_hbm, out_hbm, idx_v, rows_v, sem_g,
                       sem_w):
    wid = lax.axis_index("s") * _NC + lax.axis_index("c")
    base = wid * _ROWS_PER_W
    pltpu.sync_copy(idx_hbm.at[pl.ds(base, _ROWS_PER_W)], idx_v)
    gathers = [
        pltpu.async_copy(
            table_hbm.at[idx_v.at[pl.ds(c * rows_per_chunk, rows_per_chunk)]],
            rows_v.at[pl.ds(c * rows_per_chunk, rows_per_chunk)],
            sem_g,
        )
        for c in range(n_chunks)
    ]
    writes = []
    for c in range(n_chunks):
      gathers[c].wait()
      lo = base + c * rows_per_chunk
      writes += [
          pltpu.async_copy(
              rows_v.at[pl.ds(c * rows_per_chunk, rows_per_chunk)],
              out_hbm.at[b, pl.ds(lo, rows_per_chunk)],
              sem_w,
          )
          for b in range(_BATCH)
      ]
    for w in writes:
      w.wait()

  return gather_broadcast


_gather_broadcast = _make_gather_broadcast()


def kernel(x, E_absolute_position, relative_index):
  del x
  return _gather_broadcast(E_absolute_position, relative_index)
